# baseline (speedup 1.0000x reference)
import jax
import jax.numpy as jnp
from jax.experimental import pallas as pl
from jax.experimental.pallas import tpu as pltpu


def _body(adj_ref, x_ref, wt_ref, b_ref, out_ref):
    h = jnp.dot(adj_ref[...], x_ref[...], preferred_element_type=jnp.float32)
    o = jnp.dot(h, wt_ref[...], preferred_element_type=jnp.float32)
    out_ref[...] = o + b_ref[...]


def kernel(x, adj, W, b):
    n_rows, n_cols = adj.shape
    d_in = x.shape[1]
    d_out = W.shape[0]

    wt = W.T
    b2 = b.reshape(1, d_out)

    bm = 400 if n_rows % 400 == 0 else 256
    grid = (pl.cdiv(n_rows, bm),)

    return pl.pallas_call(
        _body,
        grid=grid,
        in_specs=[
            pl.BlockSpec((bm, n_cols), lambda i: (i, 0)),
            pl.BlockSpec((n_cols, d_in), lambda i: (0, 0)),
            pl.BlockSpec((d_in, d_out), lambda i: (0, 0)),
            pl.BlockSpec((1, d_out), lambda i: (0, 0)),
        ],
        out_specs=pl.BlockSpec((bm, d_out), lambda i: (i, 0)),
        out_shape=jax.ShapeDtypeStruct((n_rows, d_out), jnp.float32),
        compiler_params=pltpu.CompilerParams(
            dimension_semantics=("parallel",),
            vmem_limit_bytes=64 * 1024 * 1024,
        ),
    )(adj, x, wt, b2)


# speedup vs baseline: 1.0166x; 1.0166x over previous
import jax
import jax.numpy as jnp
from jax.experimental import pallas as pl
from jax.experimental.pallas import tpu as pltpu


def _body(adj_ref, x_ref, wt_ref, b_ref, out_ref):
    h = jnp.dot(adj_ref[...], x_ref[...], preferred_element_type=jnp.float32)
    o = jnp.dot(h, wt_ref[...], preferred_element_type=jnp.float32)
    out_ref[...] = o + b_ref[...]


def kernel(x, adj, W, b):
    n_rows, n_cols = adj.shape
    d_in = x.shape[1]
    d_out = W.shape[0]

    wt = W.T
    b2 = b.reshape(1, d_out)

    bm = 504 if n_rows % 8 == 0 else 256
    grid = (pl.cdiv(n_rows, bm),)

    return pl.pallas_call(
        _body,
        grid=grid,
        in_specs=[
            pl.BlockSpec((bm, n_cols), lambda i: (i, 0)),
            pl.BlockSpec((n_cols, d_in), lambda i: (0, 0)),
            pl.BlockSpec((d_in, d_out), lambda i: (0, 0)),
            pl.BlockSpec((1, d_out), lambda i: (0, 0)),
        ],
        out_specs=pl.BlockSpec((bm, d_out), lambda i: (i, 0)),
        out_shape=jax.ShapeDtypeStruct((n_rows, d_out), jnp.float32),
        compiler_params=pltpu.CompilerParams(
            dimension_semantics=("parallel",),
            vmem_limit_bytes=64 * 1024 * 1024,
        ),
    )(adj, x, wt, b2)
